# Initial kernel scaffold; baseline (speedup 1.0000x reference)
import jax
import jax.numpy as jnp
from jax.experimental import pallas as pl


def kernel(pos, batch, idx1, row1, col1, idx2, row2, col2, W1a, b1a, W1b, b1b, W1c, b1c, W2a, b2a, W2b, b2b, W2c, b2c, Wl1, bl1, Wl2, bl2, Wl3, bl3):
    raise NotImplementedError("write your pallas kernel here")


# speedup vs baseline: 1.6613x; 1.6613x over previous
import functools

import jax
import jax.numpy as jnp
from jax import lax
from jax.experimental import pallas as pl
from jax.experimental.pallas import tpu as pltpu
from jax.experimental.pallas import tpu_sc as plsc

N, N1, E1, N2, E2 = 10000, 5000, 320000, 1250, 80000
NC, NS, L = 2, 16, 16
NW = NC * NS
E1P = E1 + 512
E2P = 81920
SEG1 = 160
N1P = SEG1 * NW
SEG2 = 40
N2P = SEG2 * NW
CB1 = 512
CB2 = 192
BB = 512
A1B = 1000
A2B = 512
GD = E2P // NW
DCH = 512
NPF = 30016


def _mesh():
    return plsc.VectorSubcoreMesh(
        core_axis_name="c", subcore_axis_name="s",
        num_cores=NC, num_subcores=NS)


def _wid():
    return lax.axis_index("s") * NC + lax.axis_index("c")


def _gather_pdiff_body(posf_h, idx1_h, row1_h, col1_h, idx2_h, row2_h,
                       col2_h, pd1_h, pd2_h,
                       posf_v, idx1_v, idx2_v, row_v, col_v, pd1_v, pd2_v):
    w = _wid()
    pltpu.sync_copy(posf_h, posf_v)
    pltpu.sync_copy(idx1_h, idx1_v)
    pltpu.sync_copy(idx2_h, idx2_v)

    base1 = w * (E1 // NW)

    def l1_blk(blk, _):
        e0 = base1 + blk * A1B
        pltpu.sync_copy(row1_h.at[pl.ds(e0, A1B)], row_v.at[pl.ds(0, A1B)])
        pltpu.sync_copy(col1_h.at[pl.ds(e0, A1B)], col_v.at[pl.ds(0, A1B)])

        def inner(e, _):
            c = col_v[pl.ds(e, L)][0]
            pj = posf_v[pl.ds(c * 3, L)]
            r = row_v[pl.ds(e, L)][0]
            i1 = idx1_v[pl.ds(r, L)][0]
            pi = posf_v[pl.ds(i1 * 3, L)]
            pd1_v[pl.ds(e * L, L)] = pj - pi
            return 0

        lax.fori_loop(0, A1B, inner, 0)
        pltpu.sync_copy(pd1_v, pd1_h.at[pl.ds(e0 * L, A1B * L)])
        return 0

    lax.fori_loop(0, (E1 // NW) // A1B, l1_blk, 0)

    base2 = w * (E2P // NW)

    def l2_blk(blk, _):
        e0 = base2 + blk * A2B
        pltpu.sync_copy(row2_h.at[pl.ds(e0, A2B)], row_v.at[pl.ds(0, A2B)])
        pltpu.sync_copy(col2_h.at[pl.ds(e0, A2B)], col_v.at[pl.ds(0, A2B)])

        def inner(e, _):
            c = col_v[pl.ds(e, L)][0]
            a = idx1_v[pl.ds(c, L)][0]
            pj = posf_v[pl.ds(a * 3, L)]
            r = jnp.minimum(row_v[pl.ds(e, L)][0], N2 - 1)
            b = idx2_v[pl.ds(r, L)][0]
            bb = idx1_v[pl.ds(b, L)][0]
            pi = posf_v[pl.ds(bb * 3, L)]
            pd2_v[pl.ds(e * L, L)] = pj - pi
            return 0

        lax.fori_loop(0, A2B, inner, 0)
        pltpu.sync_copy(pd2_v, pd2_h.at[pl.ds(e0 * L, A2B * L)])
        return 0

    lax.fori_loop(0, (E2P // NW) // A2B, l2_blk, 0)


def _gather_pdiff(posf, idx1p, row1, col1, idx2p, row2p, col2p):
    f = pl.kernel(
        _gather_pdiff_body,
        out_type=(jax.ShapeDtypeStruct((E1P * L,), jnp.float32),
                  jax.ShapeDtypeStruct((E2P * L,), jnp.float32)),
        mesh=_mesh(),
        scratch_types=[
            pltpu.VMEM((NPF,), jnp.float32),
            pltpu.VMEM((N1 + 2 * L,), jnp.int32),
            pltpu.VMEM((N2 + 2 * L,), jnp.int32),
            pltpu.VMEM((A1B + L,), jnp.int32),
            pltpu.VMEM((A1B + L,), jnp.int32),
            pltpu.VMEM((A1B * L,), jnp.float32),
            pltpu.VMEM((A2B * L,), jnp.float32),
        ],
    )
    return f(posf, idx1p, row1, col1, idx2p, row2p, col2p)


def _mlp1_body(pd, wa, ba, wb, bb_, wc, bc, out):
    pdm = pd[...]
    h = jnp.dot(pdm, wa[...], preferred_element_type=jnp.float32) + ba[...]
    h = jnp.maximum(h, 0.0)
    h = jnp.dot(h, wb[...], preferred_element_type=jnp.float32) + bb_[...]
    h = jnp.maximum(h, 0.0)
    out[...] = jnp.dot(h, wc[...], preferred_element_type=jnp.float32) + bc[...]


def _mlp1(pd1, W1a_p, b1a, W1b, b1b, W1c, b1c):
    return pl.pallas_call(
        _mlp1_body,
        grid=(E1P // BB,),
        in_specs=[
            pl.BlockSpec((BB, L), lambda i: (i, 0)),
            pl.BlockSpec((L, 64), lambda i: (0, 0)),
            pl.BlockSpec((1, 64), lambda i: (0, 0)),
            pl.BlockSpec((64, 64), lambda i: (0, 0)),
            pl.BlockSpec((1, 64), lambda i: (0, 0)),
            pl.BlockSpec((64, 128), lambda i: (0, 0)),
            pl.BlockSpec((1, 128), lambda i: (0, 0)),
        ],
        out_specs=pl.BlockSpec((BB, 128), lambda i: (i, 0)),
        out_shape=jax.ShapeDtypeStruct((E1P, 128), jnp.float32),
    )(pd1.reshape(E1P, L), W1a_p, b1a.reshape(1, 64), W1b, b1b.reshape(1, 64),
      W1c, b1c.reshape(1, 128))


def _mlp2_body(xg, pd, wax, wap, ba, wb, bb_, wc, bc, out):
    pdm = pd[...]
    h = (jnp.dot(xg[...], wax[...], preferred_element_type=jnp.float32)
         + jnp.dot(pdm, wap[...], preferred_element_type=jnp.float32)
         + ba[...])
    h = jnp.maximum(h, 0.0)
    h = jnp.dot(h, wb[...], preferred_element_type=jnp.float32) + bb_[...]
    h = jnp.maximum(h, 0.0)
    out[...] = jnp.dot(h, wc[...], preferred_element_type=jnp.float32) + bc[...]


def _mlp2(xg, pd2, W2ax, W2ap, b2a, W2b, b2b, W2c, b2c):
    return pl.pallas_call(
        _mlp2_body,
        grid=(E2P // BB,),
        in_specs=[
            pl.BlockSpec((BB, 128), lambda i: (i, 0)),
            pl.BlockSpec((BB, L), lambda i: (i, 0)),
            pl.BlockSpec((128, 128), lambda i: (0, 0)),
            pl.BlockSpec((L, 128), lambda i: (0, 0)),
            pl.BlockSpec((1, 128), lambda i: (0, 0)),
            pl.BlockSpec((128, 128), lambda i: (0, 0)),
            pl.BlockSpec((1, 128), lambda i: (0, 0)),
            pl.BlockSpec((128, 256), lambda i: (0, 0)),
            pl.BlockSpec((1, 256), lambda i: (0, 0)),
        ],
        out_specs=pl.BlockSpec((BB, 256), lambda i: (i, 0)),
        out_shape=jax.ShapeDtypeStruct((E2P, 256), jnp.float32),
    )(xg, pd2.reshape(E2P, L), W2ax, W2ap, b2a.reshape(1, 128),
      W2b, b2b.reshape(1, 128), W2c, b2c.reshape(1, 256))


def _segmax_body(m_h, rows_h, bnd_h, x_h, m_v, out_v, rows_v, bnd_v, cur_s,
                 *, F, CB, SEGS, NV):
    w = _wid()
    pltpu.sync_copy(bnd_h.at[pl.ds(w * 8, 8)], bnd_v.at[pl.ds(0, 8)])
    t = bnd_v[pl.ds(0, L)]
    s0 = pl.multiple_of(t[0], 8)
    s1 = t[1]
    e0 = pl.multiple_of(t[2], 8)
    nblk = t[3]
    zer = jnp.zeros((L,), jnp.float32)
    ninf = jnp.full((L,), -jnp.inf, jnp.float32)

    def zb(i, _):
        out_v[pl.ds(i * L, L)] = zer
        return 0

    lax.fori_loop(0, SEGS * NV, zb, 0)

    cur_s[0] = -1

    def flush(cur, acc):
        s = cur - s0
        for k in range(NV):
            out_v[pl.ds(s * F + k * L, L)] = acc[k]

    def blk(b, acc0):
        eb = pl.multiple_of(e0 + b * CB, 8)
        pltpu.sync_copy(m_h.at[pl.ds(eb * F, CB * F)], m_v)
        pltpu.sync_copy(rows_h.at[pl.ds(eb, CB)], rows_v.at[pl.ds(0, CB)])

        def ebody(e, acc):
            seg = rows_v[pl.ds(e, L)][0]
            cur = cur_s[0]
            is_new = seg != cur

            @pl.when(is_new & (cur >= s0) & (cur < s1))
            def _():
                flush(cur, acc)

            acc = tuple(jnp.where(is_new, ninf, acc[k]) for k in range(NV))
            acc = tuple(
                jnp.maximum(acc[k], m_v[pl.ds(e * F + k * L, L)])
                for k in range(NV))
            cur_s[0] = seg
            return acc

        return lax.fori_loop(0, CB, ebody, acc0)

    acc = lax.fori_loop(0, nblk, blk, (ninf,) * NV)
    cur = cur_s[0]

    @pl.when((cur >= s0) & (cur < s1))
    def _():
        flush(cur, acc)

    pltpu.sync_copy(out_v, x_h.at[pl.ds(s0 * F, SEGS * F)])


def _segmax(m, rowsp, bounds, F, CB, SEGS, NSEGP):
    NV = F // L
    body = functools.partial(_segmax_body, F=F, CB=CB, SEGS=SEGS, NV=NV)
    f = pl.kernel(
        body,
        out_type=jax.ShapeDtypeStruct((NSEGP * F,), jnp.float32),
        mesh=_mesh(),
        scratch_types=[
            pltpu.VMEM((CB * F,), jnp.float32),
            pltpu.VMEM((SEGS * F,), jnp.float32),
            pltpu.VMEM((CB + L,), jnp.int32),
            pltpu.VMEM((L,), jnp.int32),
            pltpu.SMEM((8,), jnp.int32),
        ],
    )
    return f(m.reshape(-1), rowsp, bounds).reshape(NSEGP, F)


def _bounds(rows, nseg_total, segs_per_w, cb):
    w = jnp.arange(NW, dtype=jnp.int32)
    s0 = w * segs_per_w
    s1 = s0 + segs_per_w
    e0 = jnp.searchsorted(rows, jnp.minimum(s0, nseg_total),
                          side="left").astype(jnp.int32)
    e0 = (e0 // 8) * 8
    ee = jnp.searchsorted(rows, jnp.minimum(s1, nseg_total),
                          side="left").astype(jnp.int32)
    nblk = jnp.where(ee > e0, (ee - e0 + cb - 1) // cb, 0).astype(jnp.int32)
    z = jnp.zeros_like(s0)
    return jnp.stack([s0, s1, e0, nblk, z, z, z, z], axis=1).reshape(-1)


def _gatherx_body(x1_h, col2_h, xg_h, idx_v, buf_v, sem):
    w = _wid()
    base = w * GD
    for c in range(GD // DCH):
        pltpu.sync_copy(col2_h.at[pl.ds(base + c * DCH, DCH)], idx_v)
        pltpu.async_copy(x1_h.at[idx_v], buf_v, sem).wait()
        pltpu.sync_copy(buf_v, xg_h.at[pl.ds(base + c * DCH, DCH)])


def _gather_x1(x1, col2p):
    f = pl.kernel(
        _gatherx_body,
        out_type=jax.ShapeDtypeStruct((E2P, 128), jnp.float32),
        mesh=_mesh(),
        scratch_types=[
            pltpu.VMEM((DCH,), jnp.int32),
            pltpu.VMEM((DCH, 128), jnp.float32),
            pltpu.SemaphoreType.DMA,
        ],
    )
    return f(x1, col2p)


def _head_body(x, w1, b1, w2, b2, w3r, b3, out):
    h = jnp.dot(x[...], w1[...], preferred_element_type=jnp.float32) + b1[...]
    h = jnp.maximum(h, 0.0)
    h = jnp.dot(h, w2[...], preferred_element_type=jnp.float32) + b2[...]
    h = jnp.maximum(h, 0.0)
    t = jnp.sum(h * w3r[...], axis=1, keepdims=True) + b3[...]
    out[...] = jax.nn.sigmoid(t)


def _head(x2, Wl1, bl1, Wl2, bl2, Wl3, bl3):
    return pl.pallas_call(
        _head_body,
        grid=(1,),
        in_specs=[
            pl.BlockSpec((N2P, 256), lambda i: (0, 0)),
            pl.BlockSpec((256, 256), lambda i: (0, 0)),
            pl.BlockSpec((1, 256), lambda i: (0, 0)),
            pl.BlockSpec((256, 128), lambda i: (0, 0)),
            pl.BlockSpec((1, 128), lambda i: (0, 0)),
            pl.BlockSpec((1, 128), lambda i: (0, 0)),
            pl.BlockSpec((1, 1), lambda i: (0, 0)),
        ],
        out_specs=pl.BlockSpec((N2P, 1), lambda i: (0, 0)),
        out_shape=jax.ShapeDtypeStruct((N2P, 1), jnp.float32),
    )(x2, Wl1, bl1.reshape(1, 256), Wl2, bl2.reshape(1, 128),
      Wl3.reshape(1, 128), bl3.reshape(1, 1))


def kernel(pos, batch, idx1, row1, col1, idx2, row2, col2,
           W1a, b1a, W1b, b1b, W1c, b1c,
           W2a, b2a, W2b, b2b, W2c, b2c,
           Wl1, bl1, Wl2, bl2, Wl3, bl3):
    f32 = jnp.float32
    i32 = jnp.int32
    posf = jnp.concatenate([pos.reshape(-1),
                            jnp.zeros((NPF - 3 * N,), f32)])
    idx1p = jnp.concatenate([idx1, jnp.zeros((2 * L,), i32)])
    idx2p = jnp.concatenate([idx2, jnp.zeros((2 * L,), i32)])
    row1p = jnp.concatenate([row1, jnp.full((E1P - E1,), N1, i32)])
    row2p = jnp.concatenate([row2, jnp.full((E2P - E2,), N2, i32)])
    col2p = jnp.concatenate([col2, jnp.zeros((E2P - E2,), i32)])

    pd1, pd2 = _gather_pdiff(posf, idx1p, row1, col1, idx2p, row2p, col2p)

    W1a_p = jnp.concatenate([W1a, jnp.zeros((L - 3, 64), f32)], axis=0)
    m1 = _mlp1(pd1, W1a_p, b1a, W1b, b1b, W1c, b1c)
    x1 = _segmax(m1, row1p, _bounds(row1, N1, SEG1, CB1),
                 F=128, CB=CB1, SEGS=SEG1, NSEGP=N1P)

    xg = _gather_x1(x1, col2p)
    W2ax = W2a[:128]
    W2ap = jnp.concatenate([W2a[128:], jnp.zeros((L - 3, 128), f32)], axis=0)
    m2 = _mlp2(xg, pd2, W2ax, W2ap, b2a, W2b, b2b, W2c, b2c)
    x2 = _segmax(m2, row2p, _bounds(row2, N2, SEG2, CB2),
                 F=256, CB=CB2, SEGS=SEG2, NSEGP=N2P)

    y = _head(x2, Wl1, bl1, Wl2, bl2, Wl3, bl3)
    return y[:N2]
